# baseline (speedup 1.0000x reference)
import jax
import jax.numpy as jnp
from jax import lax
from jax.experimental import pallas as pl
from jax.experimental.pallas import tpu as pltpu
from jax.experimental.pallas import tpu_sc as plsc

N = 10000
E = 320000
D = 128
H = 500
HP = 512
W = 256
NC = 2
NS = 16
EPT = E // NS
G = 800
GV = G // 16
B = 64
CAP = G + 2 * B + 48
SPC = 512
CAP2 = SPC + G + B
SPE = E + SPC
RW = 5120
NP = 2
RPT = RW // NS
NR = NP * RW
RB = 1000
GRID = N // RB

_f32 = jnp.float32
_i32 = jnp.int32


def _enc_body(x_ref, wenc_ref, benc_ref, o_ref):
    xb = x_ref[...]
    hb = jax.nn.sigmoid(
        lax.dot_general(xb, wenc_ref[...], (((1,), (1,)), ((), ())),
                        preferred_element_type=_f32) + benc_ref[...])
    hp = jnp.concatenate([hb, jnp.zeros((RB, HP - H), _f32)], axis=1)
    o_ref[...] = hp.reshape(2 * RB, W)


def _encoder(x, wenc, benc2):
    return pl.pallas_call(
        _enc_body,
        grid=(GRID,),
        in_specs=[
            pl.BlockSpec((RB, D), lambda i: (i, 0)),
            pl.BlockSpec((H, D), lambda i: (0, 0)),
            pl.BlockSpec((1, H), lambda i: (0, 0)),
        ],
        out_specs=pl.BlockSpec((2 * RB, W), lambda i: (i, 0)),
        out_shape=jax.ShapeDtypeStruct((2 * N, W), _f32),
    )(x, wenc, benc2)


def _fold_body(wn_ref, wr_ref, wdT_ref, w1_ref, w2_ref):
    wdT = wdT_ref[...]
    w1 = lax.dot_general(wn_ref[...], wdT, (((1,), (0,)), ((), ())),
                         preferred_element_type=_f32)
    w2 = lax.dot_general(wr_ref[...], wdT, (((0,), (0,)), ((), ())),
                         preferred_element_type=_f32)
    z = jnp.zeros((HP - H, D), _f32)
    w1_ref[...] = jnp.concatenate([w1, z], axis=0)
    w2_ref[...] = jnp.concatenate([w2, z], axis=0)


def _fold(w_neigh, w_root, wdecT):
    return pl.pallas_call(
        _fold_body,
        out_shape=(jax.ShapeDtypeStruct((HP, D), _f32),
                   jax.ShapeDtypeStruct((HP, D), _f32)),
    )(w_neigh, w_root, wdecT)


NGALL = E // G


def _sc_body(esrc_hbm, edst_hbm, ew_hbm, h_hbm,
             out_hbm, sp_src, sp_loc, sp_w,
             acc, src_s0, src_s1, dst_s0, dst_s1, w_s0, w_s1,
             src_c, loc_c, w_c, src_c2, loc_c2, w_c2, rows0, rows1,
             ssem0, ssem1, gsem0, gsem1):
    c = lax.axis_index("c")
    tid = lax.axis_index("s")
    iot = lax.iota(_i32, 16)
    z16 = jnp.zeros((16,), _f32)
    ssems = (ssem0, ssem1)
    gsems = (gsem0, gsem1)
    src_bufs = (src_s0, src_s1)
    dst_bufs = (dst_s0, dst_s1)
    w_bufs = (w_s0, w_s1)
    row_bufs = (rows0, rows1)

    def zero_acc(i, carry):
        for k2 in range(W // 16):
            acc[i, pl.ds(k2 * 16, 16)] = z16
        return carry

    def stage_issue(g, buf):
        gb = pl.multiple_of(g * G, 8)
        sem = ssems[buf]
        pltpu.async_copy(esrc_hbm.at[pl.ds(gb, G)], src_bufs[buf], sem)
        pltpu.async_copy(edst_hbm.at[pl.ds(gb, G)], dst_bufs[buf], sem)
        pltpu.async_copy(ew_hbm.at[pl.ds(gb, G)], w_bufs[buf], sem)

    def stage_wait(g, buf):
        gb = pl.multiple_of(g * G, 8)
        sem = ssems[buf]
        pltpu.make_async_copy(esrc_hbm.at[pl.ds(gb, G)], src_bufs[buf], sem).wait()
        pltpu.make_async_copy(edst_hbm.at[pl.ds(gb, G)], dst_bufs[buf], sem).wait()
        pltpu.make_async_copy(ew_hbm.at[pl.ds(gb, G)], w_bufs[buf], sem).wait()

    def gather_issue(boff, rbuf):
        pltpu.async_copy(h_hbm.at[src_c.at[pl.ds(boff, B)]],
                         row_bufs[rbuf], gsems[rbuf])

    def gather_wait(boff, rbuf):
        pltpu.make_async_copy(h_hbm.at[src_c.at[pl.ds(boff, B)]],
                              row_bufs[rbuf], gsems[rbuf]).wait()

    def accum(boff, rbuf):
        rows = row_bufs[rbuf]
        for sub in range(B // 16):
            w16 = w_c[pl.ds(boff + sub * 16, 16)]
            loc16 = loc_c[pl.ds(boff + sub * 16, 16)]
            rid = iot + sub * 16

            def cbody(j8, cc, rid=rid, w16=w16, loc16=loc16):
                for k2 in range(8):
                    cid = jnp.zeros((16,), _i32) + (j8 * 8 + k2)
                    v = plsc.load_gather(rows, [rid, cid])
                    plsc.addupdate_scatter(acc, [loc16, cid], v * w16)
                return cc
            lax.fori_loop(0, W // 8, cbody, 0)

    def drain_batches(cnt):
        nb = cnt // B

        @pl.when(nb > 0)
        def _():
            gather_issue(0, 0)

        def bb2(b2, carry):
            b0 = b2 * 2
            b1 = b0 + 1
            o0 = pl.multiple_of(b0 * B, B)
            o1 = pl.multiple_of(b1 * B, B)
            o2 = pl.multiple_of((b1 + 1) * B, B)

            @pl.when(b0 < nb)
            def _():
                gather_wait(o0, 0)

                @pl.when(b1 < nb)
                def _():
                    gather_issue(o1, 1)
                accum(o0, 0)

            @pl.when(b1 < nb)
            def _():
                gather_wait(o1, 1)

                @pl.when(b1 + 1 < nb)
                def _():
                    gather_issue(o2, 0)
                accum(o1, 1)
            return carry
        lax.fori_loop(0, (nb + 1) // 2, bb2, 0)
        left = cnt - nb * B
        base = pl.multiple_of(nb * B, B)
        for t in range(B // 16):
            sl_src = pl.ds(base + t * 16, 16)
            sl_dst = pl.ds(t * 16, 16)
            src_c[sl_dst] = src_c[sl_src]
            loc_c[sl_dst] = loc_c[sl_src]
            w_c[sl_dst] = w_c[sl_src]
        return left

    def pad_tail(cnt):
        for t in range(B // 16):
            sl = pl.ds(cnt + t * 16, 16)
            src_c[sl] = iot + t * 16
            loc_c[sl] = iot + t * 16
            w_c[sl] = z16

    lo0 = tid * RPT
    lo1 = RW + tid * RPT

    def fbody_gen(buf, lo_a, lo_b):
        def fbody(i, carry):
            cnt, cnt2 = carry
            off = pl.multiple_of(i * 16, 16)
            d16 = dst_bufs[buf][pl.ds(off, 16)]
            m0 = (d16 >= lo_a) & (d16 < lo_a + RPT)
            m1 = (d16 >= lo_b) & (d16 < lo_b + RPT)
            s16 = src_bufs[buf][pl.ds(off, 16)] * 2 + c
            wv = w_bufs[buf][pl.ds(off, 16)]
            pos = plsc.cumsum(m0.astype(_i32)) - 1 + cnt
            plsc.store_scatter(src_c, [pos], s16, mask=m0)
            plsc.store_scatter(loc_c, [pos], d16 - lo_a, mask=m0)
            plsc.store_scatter(w_c, [pos], wv, mask=m0)
            pos2 = plsc.cumsum(m1.astype(_i32)) - 1 + cnt2
            plsc.store_scatter(src_c2, [pos2], s16, mask=m1)
            plsc.store_scatter(loc_c2, [pos2], d16 - lo_b, mask=m1)
            plsc.store_scatter(w_c2, [pos2], wv, mask=m1)
            return (cnt + plsc.all_reduce_population_count(m0),
                    cnt2 + plsc.all_reduce_population_count(m1))
        return fbody

    def drain_spill(cnt2, snb):
        nchunks = cnt2 // SPC

        def scw(j, snb_):
            o = pl.multiple_of(j * SPC, 8)
            so = pl.multiple_of(snb_ * SPC, 8)
            pltpu.sync_copy(src_c2.at[pl.ds(o, SPC)],
                            sp_src.at[c, tid, pl.ds(so, SPC)])
            pltpu.sync_copy(loc_c2.at[pl.ds(o, SPC)],
                            sp_loc.at[c, tid, pl.ds(so, SPC)])
            pltpu.sync_copy(w_c2.at[pl.ds(o, SPC)],
                            sp_w.at[c, tid, pl.ds(so, SPC)])
            return snb_ + 1
        snb = lax.fori_loop(0, nchunks, scw, snb)
        base = pl.multiple_of(nchunks * SPC, 8)
        for t in range(SPC // 16):
            sl_src = pl.ds(base + t * 16, 16)
            sl_dst = pl.ds(t * 16, 16)
            src_c2[sl_dst] = src_c2[sl_src]
            loc_c2[sl_dst] = loc_c2[sl_src]
            w_c2[sl_dst] = w_c2[sl_src]
        return cnt2 - nchunks * SPC, snb

    lax.fori_loop(0, RPT, zero_acc, 0)
    fb0 = fbody_gen(0, lo0, lo1)
    fb1 = fbody_gen(1, lo0, lo1)
    zi16 = jnp.zeros((16,), _i32)
    stage_issue(0, 0)

    def gbody2(g2, carry):
        cntv, cnt2v, snb = carry
        g0 = g2 * 2
        g1 = g0 + 1
        stage_wait(g0, 0)
        stage_issue(g1, 1)
        cntv, cnt2v = lax.fori_loop(0, GV, fb0, (cntv, cnt2v))
        cnt = drain_batches(cntv[0])
        cnt2, snb = drain_spill(cnt2v[0], snb)
        cntv = zi16 + cnt
        cnt2v = zi16 + cnt2
        stage_wait(g1, 1)

        @pl.when(g1 + 1 < NGALL)
        def _():
            stage_issue(g1 + 1, 0)
        cntv, cnt2v = lax.fori_loop(0, GV, fb1, (cntv, cnt2v))
        cnt = drain_batches(cntv[0])
        cnt2, snb = drain_spill(cnt2v[0], snb)
        return (zi16 + cnt, zi16 + cnt2, snb)

    cntv, cnt2v, snb = lax.fori_loop(0, NGALL // 2, gbody2, (zi16, zi16, 0))
    cnt = cntv[0]
    cnt2 = cnt2v[0]
    pad_tail(cnt)
    gather_issue(0, 0)
    gather_wait(0, 0)
    accum(0, 0)
    for t in range(SPC // 16):
        sl = pl.ds(cnt2 + t * 16, 16)
        src_c2[sl] = iot + t * 16
        loc_c2[sl] = iot + t * 16
        w_c2[sl] = z16
    so = pl.multiple_of(snb * SPC, 8)
    pltpu.sync_copy(src_c2.at[pl.ds(0, SPC)], sp_src.at[c, tid, pl.ds(so, SPC)])
    pltpu.sync_copy(loc_c2.at[pl.ds(0, SPC)], sp_loc.at[c, tid, pl.ds(so, SPC)])
    pltpu.sync_copy(w_c2.at[pl.ds(0, SPC)], sp_w.at[c, tid, pl.ds(so, SPC)])
    ns = snb + 1
    pltpu.sync_copy(acc, out_hbm.at[c, pl.ds(lo0, RPT)])

    lax.fori_loop(0, RPT, zero_acc, 0)

    def sbody(sidx, carry):
        off = pl.multiple_of(sidx * SPC, 8)
        pltpu.sync_copy(sp_src.at[c, tid, pl.ds(off, SPC)],
                        src_c.at[pl.ds(0, SPC)])
        pltpu.sync_copy(sp_loc.at[c, tid, pl.ds(off, SPC)],
                        loc_c.at[pl.ds(0, SPC)])
        pltpu.sync_copy(sp_w.at[c, tid, pl.ds(off, SPC)],
                        w_c.at[pl.ds(0, SPC)])
        drain_batches(SPC)
        return carry

    lax.fori_loop(0, ns, sbody, 0)
    pltpu.sync_copy(acc, out_hbm.at[c, pl.ds(lo1, RPT)])


def _sc_segment_sum(edge_index, edge_weight, h_flat):
    mesh = plsc.VectorSubcoreMesh(core_axis_name="c", subcore_axis_name="s",
                                  num_cores=NC, num_subcores=NS)
    out = pl.kernel(
        _sc_body,
        out_type=(jax.ShapeDtypeStruct((NC, NR, W), _f32),
                  jax.ShapeDtypeStruct((NC, NS, SPE), _i32),
                  jax.ShapeDtypeStruct((NC, NS, SPE), _i32),
                  jax.ShapeDtypeStruct((NC, NS, SPE), _f32)),
        mesh=mesh,
        compiler_params=pltpu.CompilerParams(needs_layout_passes=False),
        scratch_types=[
            pltpu.VMEM((RPT, W), _f32),
            pltpu.VMEM((G,), _i32),
            pltpu.VMEM((G,), _i32),
            pltpu.VMEM((G,), _i32),
            pltpu.VMEM((G,), _i32),
            pltpu.VMEM((G,), _f32),
            pltpu.VMEM((G,), _f32),
            pltpu.VMEM((CAP,), _i32),
            pltpu.VMEM((CAP,), _i32),
            pltpu.VMEM((CAP,), _f32),
            pltpu.VMEM((CAP2,), _i32),
            pltpu.VMEM((CAP2,), _i32),
            pltpu.VMEM((CAP2,), _f32),
            pltpu.VMEM((B, W), _f32),
            pltpu.VMEM((B, W), _f32),
            pltpu.SemaphoreType.DMA,
            pltpu.SemaphoreType.DMA,
            pltpu.SemaphoreType.DMA,
            pltpu.SemaphoreType.DMA,
        ],
    )(edge_index[0], edge_index[1], edge_weight, h_flat)
    return out[0]


def _comb_body(hf_ref, s_ref, w1_ref, w2_ref, bd_ref, o_ref):
    hb = hf_ref[...].reshape(RB, HP)
    sb = jnp.concatenate([s_ref[0], s_ref[1]], axis=1)
    o_ref[...] = (
        lax.dot_general(sb, w1_ref[...], (((1,), (0,)), ((), ())),
                        preferred_element_type=_f32)
        + lax.dot_general(hb, w2_ref[...], (((1,), (0,)), ((), ())),
                          preferred_element_type=_f32)
        + bd_ref[...])


def _combine(h_flat, s_st, w1, w2, bdec2):
    return pl.pallas_call(
        _comb_body,
        grid=(GRID,),
        in_specs=[
            pl.BlockSpec((2 * RB, W), lambda i: (i, 0)),
            pl.BlockSpec((NC, RB, W), lambda i: (0, i, 0)),
            pl.BlockSpec((HP, D), lambda i: (0, 0)),
            pl.BlockSpec((HP, D), lambda i: (0, 0)),
            pl.BlockSpec((1, D), lambda i: (0, 0)),
        ],
        out_specs=pl.BlockSpec((RB, D), lambda i: (i, 0)),
        out_shape=jax.ShapeDtypeStruct((N, D), _f32),
    )(h_flat, s_st, w1, w2, bdec2)


def kernel(x, edge_index, edge_weight, wenc, benc, w_neigh, w_root, wdec, bdec):
    h_flat = _encoder(x, wenc, benc.reshape(1, H))
    w1, w2 = _fold(w_neigh, w_root, wdec.T)
    s_st = _sc_segment_sum(edge_index, edge_weight, h_flat)
    return _combine(h_flat, s_st, w1, w2, bdec.reshape(1, D))


# speedup vs baseline: 3.2458x; 3.2458x over previous
import jax
import jax.numpy as jnp
from jax import lax
from jax.experimental import pallas as pl
from jax.experimental.pallas import tpu as pltpu
from jax.experimental.pallas import tpu_sc as plsc

N = 10000
E = 320000
D = 128
H = 500
HP = 512
W = 256
NC = 2
NS = 16
EPT = E // NS
G = 800
GV = G // 16
B = 64
CAP = G + 2 * B + 48
SPC = 512
CAP2 = SPC + G + B
SPE = E + SPC
RW = 5120
NP = 2
RPT = RW // NS
NR = NP * RW
RB = 1000
GRID = N // RB

_f32 = jnp.float32
_i32 = jnp.int32


def _enc_body(x_ref, wenc_ref, benc_ref, o_ref):
    xb = x_ref[...]
    hb = jax.nn.sigmoid(
        lax.dot_general(xb, wenc_ref[...], (((1,), (1,)), ((), ())),
                        preferred_element_type=_f32) + benc_ref[...])
    hp = jnp.concatenate([hb, jnp.zeros((RB, HP - H), _f32)], axis=1)
    o_ref[...] = hp.reshape(2 * RB, W)


def _encoder(x, wenc, benc2):
    return pl.pallas_call(
        _enc_body,
        grid=(GRID,),
        in_specs=[
            pl.BlockSpec((RB, D), lambda i: (i, 0)),
            pl.BlockSpec((H, D), lambda i: (0, 0)),
            pl.BlockSpec((1, H), lambda i: (0, 0)),
        ],
        out_specs=pl.BlockSpec((2 * RB, W), lambda i: (i, 0)),
        out_shape=jax.ShapeDtypeStruct((2 * N, W), _f32),
    )(x, wenc, benc2)


def _fold_body(wn_ref, wr_ref, wdT_ref, w1_ref, w2_ref):
    wdT = wdT_ref[...]
    w1 = lax.dot_general(wn_ref[...], wdT, (((1,), (0,)), ((), ())),
                         preferred_element_type=_f32)
    w2 = lax.dot_general(wr_ref[...], wdT, (((0,), (0,)), ((), ())),
                         preferred_element_type=_f32)
    z = jnp.zeros((HP - H, D), _f32)
    w1_ref[...] = jnp.concatenate([w1, z], axis=0)
    w2_ref[...] = jnp.concatenate([w2, z], axis=0)


def _fold(w_neigh, w_root, wdecT):
    return pl.pallas_call(
        _fold_body,
        out_shape=(jax.ShapeDtypeStruct((HP, D), _f32),
                   jax.ShapeDtypeStruct((HP, D), _f32)),
    )(w_neigh, w_root, wdecT)


NGALL = E // G


def _sc_body(esrc_hbm, edst_hbm, ew_hbm, h_hbm,
             out_hbm, sp_src, sp_loc, sp_w,
             acc, src_s0, src_s1, dst_s0, dst_s1, w_s0, w_s1,
             src_c, loc_c, w_c, src_c2, loc_c2, w_c2, rows0, rows1,
             ssem0, ssem1, gsem0, gsem1):
    c = lax.axis_index("c")
    tid = lax.axis_index("s")
    iot = lax.iota(_i32, 16)
    z16 = jnp.zeros((16,), _f32)
    ssems = (ssem0, ssem1)
    gsems = (gsem0, gsem1)
    src_bufs = (src_s0, src_s1)
    dst_bufs = (dst_s0, dst_s1)
    w_bufs = (w_s0, w_s1)
    row_bufs = (rows0, rows1)

    def zero_acc(i, carry):
        for k2 in range(W // 16):
            acc[i, pl.ds(k2 * 16, 16)] = z16
        return carry

    def stage_issue(g, buf):
        gb = pl.multiple_of(g * G, 8)
        sem = ssems[buf]
        pltpu.async_copy(esrc_hbm.at[pl.ds(gb, G)], src_bufs[buf], sem)
        pltpu.async_copy(edst_hbm.at[pl.ds(gb, G)], dst_bufs[buf], sem)
        pltpu.async_copy(ew_hbm.at[pl.ds(gb, G)], w_bufs[buf], sem)

    def stage_wait(g, buf):
        gb = pl.multiple_of(g * G, 8)
        sem = ssems[buf]
        pltpu.make_async_copy(esrc_hbm.at[pl.ds(gb, G)], src_bufs[buf], sem).wait()
        pltpu.make_async_copy(edst_hbm.at[pl.ds(gb, G)], dst_bufs[buf], sem).wait()
        pltpu.make_async_copy(ew_hbm.at[pl.ds(gb, G)], w_bufs[buf], sem).wait()

    def gather_issue(boff, rbuf):
        pltpu.async_copy(h_hbm.at[src_c.at[pl.ds(boff, B)]],
                         row_bufs[rbuf], gsems[rbuf])

    def gather_wait(boff, rbuf):
        pltpu.make_async_copy(h_hbm.at[src_c.at[pl.ds(boff, B)]],
                              row_bufs[rbuf], gsems[rbuf]).wait()

    def accum(boff, rbuf):
        rows = row_bufs[rbuf]

        def ebody(i2, carry):
            i = i2 * 2
            lr0 = loc_c[pl.ds(boff + i, 16)][0]
            ws0 = w_c[pl.ds(boff + i, 16)][0]
            lr1 = loc_c[pl.ds(boff + i + 1, 16)][0]
            ws1 = w_c[pl.ds(boff + i + 1, 16)][0]
            for k2 in range(W // 16):
                sl = pl.ds(k2 * 16, 16)
                plsc.addupdate(acc.at[lr0, sl], rows[i, sl] * ws0)
                plsc.addupdate(acc.at[lr1, sl], rows[i + 1, sl] * ws1)
            return carry
        lax.fori_loop(0, B // 2, ebody, 0)

    def drain_batches(cnt):
        nb = cnt // B

        @pl.when(nb > 0)
        def _():
            gather_issue(0, 0)

        def bb2(b2, carry):
            b0 = b2 * 2
            b1 = b0 + 1
            o0 = pl.multiple_of(b0 * B, B)
            o1 = pl.multiple_of(b1 * B, B)
            o2 = pl.multiple_of((b1 + 1) * B, B)

            @pl.when(b0 < nb)
            def _():
                gather_wait(o0, 0)

                @pl.when(b1 < nb)
                def _():
                    gather_issue(o1, 1)
                accum(o0, 0)

            @pl.when(b1 < nb)
            def _():
                gather_wait(o1, 1)

                @pl.when(b1 + 1 < nb)
                def _():
                    gather_issue(o2, 0)
                accum(o1, 1)
            return carry
        lax.fori_loop(0, (nb + 1) // 2, bb2, 0)
        left = cnt - nb * B
        base = pl.multiple_of(nb * B, B)
        for t in range(B // 16):
            sl_src = pl.ds(base + t * 16, 16)
            sl_dst = pl.ds(t * 16, 16)
            src_c[sl_dst] = src_c[sl_src]
            loc_c[sl_dst] = loc_c[sl_src]
            w_c[sl_dst] = w_c[sl_src]
        return left

    def pad_tail(cnt):
        for t in range(B // 16):
            sl = pl.ds(cnt + t * 16, 16)
            src_c[sl] = iot + t * 16
            loc_c[sl] = iot + t * 16
            w_c[sl] = z16

    lo0 = tid * RPT
    lo1 = RW + tid * RPT

    def fbody_gen(buf, lo_a, lo_b):
        def fbody(i, carry):
            cnt, cnt2 = carry
            off = pl.multiple_of(i * 16, 16)
            d16 = dst_bufs[buf][pl.ds(off, 16)]
            m0 = (d16 >= lo_a) & (d16 < lo_a + RPT)
            m1 = (d16 >= lo_b) & (d16 < lo_b + RPT)
            s16 = src_bufs[buf][pl.ds(off, 16)] * 2 + c
            wv = w_bufs[buf][pl.ds(off, 16)]
            pos = plsc.cumsum(m0.astype(_i32)) - 1 + cnt
            plsc.store_scatter(src_c, [pos], s16, mask=m0)
            plsc.store_scatter(loc_c, [pos], d16 - lo_a, mask=m0)
            plsc.store_scatter(w_c, [pos], wv, mask=m0)
            pos2 = plsc.cumsum(m1.astype(_i32)) - 1 + cnt2
            plsc.store_scatter(src_c2, [pos2], s16, mask=m1)
            plsc.store_scatter(loc_c2, [pos2], d16 - lo_b, mask=m1)
            plsc.store_scatter(w_c2, [pos2], wv, mask=m1)
            return (cnt + plsc.all_reduce_population_count(m0),
                    cnt2 + plsc.all_reduce_population_count(m1))
        return fbody

    def drain_spill(cnt2, snb):
        nchunks = cnt2 // SPC

        def scw(j, snb_):
            o = pl.multiple_of(j * SPC, 8)
            so = pl.multiple_of(snb_ * SPC, 8)
            pltpu.sync_copy(src_c2.at[pl.ds(o, SPC)],
                            sp_src.at[c, tid, pl.ds(so, SPC)])
            pltpu.sync_copy(loc_c2.at[pl.ds(o, SPC)],
                            sp_loc.at[c, tid, pl.ds(so, SPC)])
            pltpu.sync_copy(w_c2.at[pl.ds(o, SPC)],
                            sp_w.at[c, tid, pl.ds(so, SPC)])
            return snb_ + 1
        snb = lax.fori_loop(0, nchunks, scw, snb)
        base = pl.multiple_of(nchunks * SPC, 8)
        for t in range(SPC // 16):
            sl_src = pl.ds(base + t * 16, 16)
            sl_dst = pl.ds(t * 16, 16)
            src_c2[sl_dst] = src_c2[sl_src]
            loc_c2[sl_dst] = loc_c2[sl_src]
            w_c2[sl_dst] = w_c2[sl_src]
        return cnt2 - nchunks * SPC, snb

    lax.fori_loop(0, RPT, zero_acc, 0)
    fb0 = fbody_gen(0, lo0, lo1)
    fb1 = fbody_gen(1, lo0, lo1)
    zi16 = jnp.zeros((16,), _i32)
    stage_issue(0, 0)

    def gbody2(g2, carry):
        cntv, cnt2v, snb = carry
        g0 = g2 * 2
        g1 = g0 + 1
        stage_wait(g0, 0)
        stage_issue(g1, 1)
        cntv, cnt2v = lax.fori_loop(0, GV, fb0, (cntv, cnt2v))
        cnt = drain_batches(cntv[0])
        cnt2, snb = drain_spill(cnt2v[0], snb)
        cntv = zi16 + cnt
        cnt2v = zi16 + cnt2
        stage_wait(g1, 1)

        @pl.when(g1 + 1 < NGALL)
        def _():
            stage_issue(g1 + 1, 0)
        cntv, cnt2v = lax.fori_loop(0, GV, fb1, (cntv, cnt2v))
        cnt = drain_batches(cntv[0])
        cnt2, snb = drain_spill(cnt2v[0], snb)
        return (zi16 + cnt, zi16 + cnt2, snb)

    cntv, cnt2v, snb = lax.fori_loop(0, NGALL // 2, gbody2, (zi16, zi16, 0))
    cnt = cntv[0]
    cnt2 = cnt2v[0]
    pad_tail(cnt)
    gather_issue(0, 0)
    gather_wait(0, 0)
    accum(0, 0)
    for t in range(SPC // 16):
        sl = pl.ds(cnt2 + t * 16, 16)
        src_c2[sl] = iot + t * 16
        loc_c2[sl] = iot + t * 16
        w_c2[sl] = z16
    so = pl.multiple_of(snb * SPC, 8)
    pltpu.sync_copy(src_c2.at[pl.ds(0, SPC)], sp_src.at[c, tid, pl.ds(so, SPC)])
    pltpu.sync_copy(loc_c2.at[pl.ds(0, SPC)], sp_loc.at[c, tid, pl.ds(so, SPC)])
    pltpu.sync_copy(w_c2.at[pl.ds(0, SPC)], sp_w.at[c, tid, pl.ds(so, SPC)])
    ns = snb + 1
    pltpu.sync_copy(acc, out_hbm.at[c, pl.ds(lo0, RPT)])

    lax.fori_loop(0, RPT, zero_acc, 0)

    def sbody(sidx, carry):
        off = pl.multiple_of(sidx * SPC, 8)
        pltpu.sync_copy(sp_src.at[c, tid, pl.ds(off, SPC)],
                        src_c.at[pl.ds(0, SPC)])
        pltpu.sync_copy(sp_loc.at[c, tid, pl.ds(off, SPC)],
                        loc_c.at[pl.ds(0, SPC)])
        pltpu.sync_copy(sp_w.at[c, tid, pl.ds(off, SPC)],
                        w_c.at[pl.ds(0, SPC)])
        drain_batches(SPC)
        return carry

    lax.fori_loop(0, ns, sbody, 0)
    pltpu.sync_copy(acc, out_hbm.at[c, pl.ds(lo1, RPT)])


def _sc_segment_sum(edge_index, edge_weight, h_flat):
    mesh = plsc.VectorSubcoreMesh(core_axis_name="c", subcore_axis_name="s",
                                  num_cores=NC, num_subcores=NS)
    out = pl.kernel(
        _sc_body,
        out_type=(jax.ShapeDtypeStruct((NC, NR, W), _f32),
                  jax.ShapeDtypeStruct((NC, NS, SPE), _i32),
                  jax.ShapeDtypeStruct((NC, NS, SPE), _i32),
                  jax.ShapeDtypeStruct((NC, NS, SPE), _f32)),
        mesh=mesh,
        compiler_params=pltpu.CompilerParams(needs_layout_passes=False),
        scratch_types=[
            pltpu.VMEM((RPT, W), _f32),
            pltpu.VMEM((G,), _i32),
            pltpu.VMEM((G,), _i32),
            pltpu.VMEM((G,), _i32),
            pltpu.VMEM((G,), _i32),
            pltpu.VMEM((G,), _f32),
            pltpu.VMEM((G,), _f32),
            pltpu.VMEM((CAP,), _i32),
            pltpu.VMEM((CAP,), _i32),
            pltpu.VMEM((CAP,), _f32),
            pltpu.VMEM((CAP2,), _i32),
            pltpu.VMEM((CAP2,), _i32),
            pltpu.VMEM((CAP2,), _f32),
            pltpu.VMEM((B, W), _f32),
            pltpu.VMEM((B, W), _f32),
            pltpu.SemaphoreType.DMA,
            pltpu.SemaphoreType.DMA,
            pltpu.SemaphoreType.DMA,
            pltpu.SemaphoreType.DMA,
        ],
    )(edge_index[0], edge_index[1], edge_weight, h_flat)
    return out[0]


def _comb_body(hf_ref, s_ref, w1_ref, w2_ref, bd_ref, o_ref):
    hb = hf_ref[...].reshape(RB, HP)
    sb = jnp.concatenate([s_ref[0], s_ref[1]], axis=1)
    o_ref[...] = (
        lax.dot_general(sb, w1_ref[...], (((1,), (0,)), ((), ())),
                        preferred_element_type=_f32)
        + lax.dot_general(hb, w2_ref[...], (((1,), (0,)), ((), ())),
                          preferred_element_type=_f32)
        + bd_ref[...])


def _combine(h_flat, s_st, w1, w2, bdec2):
    return pl.pallas_call(
        _comb_body,
        grid=(GRID,),
        in_specs=[
            pl.BlockSpec((2 * RB, W), lambda i: (i, 0)),
            pl.BlockSpec((NC, RB, W), lambda i: (0, i, 0)),
            pl.BlockSpec((HP, D), lambda i: (0, 0)),
            pl.BlockSpec((HP, D), lambda i: (0, 0)),
            pl.BlockSpec((1, D), lambda i: (0, 0)),
        ],
        out_specs=pl.BlockSpec((RB, D), lambda i: (i, 0)),
        out_shape=jax.ShapeDtypeStruct((N, D), _f32),
    )(h_flat, s_st, w1, w2, bdec2)


def kernel(x, edge_index, edge_weight, wenc, benc, w_neigh, w_root, wdec, bdec):
    h_flat = _encoder(x, wenc, benc.reshape(1, H))
    w1, w2 = _fold(w_neigh, w_root, wdec.T)
    s_st = _sc_segment_sum(edge_index, edge_weight, h_flat)
    return _combine(h_flat, s_st, w1, w2, bdec.reshape(1, D))


# speedup vs baseline: 3.3211x; 1.0232x over previous
import jax
import jax.numpy as jnp
from jax import lax
from jax.experimental import pallas as pl
from jax.experimental.pallas import tpu as pltpu
from jax.experimental.pallas import tpu_sc as plsc

N = 10000
E = 320000
D = 128
H = 500
HP = 512
W = 256
NC = 2
NS = 16
EPT = E // NS
G = 800
GV = G // 16
B = 64
CAP = G + 2 * B + 48
SPC = 512
CAP2 = SPC + G + B
SPE = E + SPC
RW = 5120
NP = 2
RPT = RW // NS
NR = NP * RW
RB = 1000
GRID = N // RB

_f32 = jnp.float32
_i32 = jnp.int32


def _enc_body(x_ref, wenc_ref, benc_ref, o_ref):
    xb = x_ref[...]
    hb = jax.nn.sigmoid(
        lax.dot_general(xb, wenc_ref[...], (((1,), (1,)), ((), ())),
                        preferred_element_type=_f32) + benc_ref[...])
    hp = jnp.concatenate([hb, jnp.zeros((RB, HP - H), _f32)], axis=1)
    o_ref[...] = hp.reshape(2 * RB, W)


def _encoder(x, wenc, benc2):
    return pl.pallas_call(
        _enc_body,
        grid=(GRID,),
        in_specs=[
            pl.BlockSpec((RB, D), lambda i: (i, 0)),
            pl.BlockSpec((H, D), lambda i: (0, 0)),
            pl.BlockSpec((1, H), lambda i: (0, 0)),
        ],
        out_specs=pl.BlockSpec((2 * RB, W), lambda i: (i, 0)),
        out_shape=jax.ShapeDtypeStruct((2 * N, W), _f32),
    )(x, wenc, benc2)


def _fold_body(wn_ref, wr_ref, wdT_ref, w1_ref, w2_ref):
    wdT = wdT_ref[...]
    w1 = lax.dot_general(wn_ref[...], wdT, (((1,), (0,)), ((), ())),
                         preferred_element_type=_f32)
    w2 = lax.dot_general(wr_ref[...], wdT, (((0,), (0,)), ((), ())),
                         preferred_element_type=_f32)
    z = jnp.zeros((HP - H, D), _f32)
    w1_ref[...] = jnp.concatenate([w1, z], axis=0)
    w2_ref[...] = jnp.concatenate([w2, z], axis=0)


def _fold(w_neigh, w_root, wdecT):
    return pl.pallas_call(
        _fold_body,
        out_shape=(jax.ShapeDtypeStruct((HP, D), _f32),
                   jax.ShapeDtypeStruct((HP, D), _f32)),
    )(w_neigh, w_root, wdecT)


NGALL = E // G


def _sc_body(esrc_hbm, edst_hbm, ew_hbm, h_hbm,
             out_hbm, sp_src, sp_loc, sp_w,
             acc, src_s0, src_s1, dst_s0, dst_s1, w_s0, w_s1,
             src_c, loc_c, w_c, src_c2, loc_c2, w_c2, rows0, rows1,
             ssem0, ssem1, gsem0, gsem1):
    c = lax.axis_index("c")
    tid = lax.axis_index("s")
    iot = lax.iota(_i32, 16)
    z16 = jnp.zeros((16,), _f32)
    ssems = (ssem0, ssem1)
    gsems = (gsem0, gsem1)
    src_bufs = (src_s0, src_s1)
    dst_bufs = (dst_s0, dst_s1)
    w_bufs = (w_s0, w_s1)
    row_bufs = (rows0, rows1)

    def zero_acc(i, carry):
        for k2 in range(W // 16):
            acc[i, pl.ds(k2 * 16, 16)] = z16
        return carry

    def stage_issue(g, buf):
        gb = pl.multiple_of(g * G, 8)
        sem = ssems[buf]
        pltpu.async_copy(esrc_hbm.at[pl.ds(gb, G)], src_bufs[buf], sem)
        pltpu.async_copy(edst_hbm.at[pl.ds(gb, G)], dst_bufs[buf], sem)
        pltpu.async_copy(ew_hbm.at[pl.ds(gb, G)], w_bufs[buf], sem)

    def stage_wait(g, buf):
        gb = pl.multiple_of(g * G, 8)
        sem = ssems[buf]
        pltpu.make_async_copy(esrc_hbm.at[pl.ds(gb, G)], src_bufs[buf], sem).wait()
        pltpu.make_async_copy(edst_hbm.at[pl.ds(gb, G)], dst_bufs[buf], sem).wait()
        pltpu.make_async_copy(ew_hbm.at[pl.ds(gb, G)], w_bufs[buf], sem).wait()

    def gather_issue(boff, rbuf):
        pltpu.async_copy(h_hbm.at[src_c.at[pl.ds(boff, B)]],
                         row_bufs[rbuf], gsems[rbuf])

    def gather_wait(boff, rbuf):
        pltpu.make_async_copy(h_hbm.at[src_c.at[pl.ds(boff, B)]],
                              row_bufs[rbuf], gsems[rbuf]).wait()

    def accum(boff, rbuf):
        rows = row_bufs[rbuf]

        def ebody(i4, carry):
            i = i4 * 4
            lrs = [loc_c[pl.ds(boff + i + j, 16)][0] for j in range(4)]
            wss = [w_c[pl.ds(boff + i + j, 16)][0] for j in range(4)]
            for k2 in range(W // 16):
                sl = pl.ds(k2 * 16, 16)
                for j in range(4):
                    plsc.addupdate(acc.at[lrs[j], sl], rows[i + j, sl] * wss[j])
            return carry
        lax.fori_loop(0, B // 4, ebody, 0)

    def drain_batches(cnt):
        nb = cnt // B

        @pl.when(nb > 0)
        def _():
            gather_issue(0, 0)

        def bb2(b2, carry):
            b0 = b2 * 2
            b1 = b0 + 1
            o0 = pl.multiple_of(b0 * B, B)
            o1 = pl.multiple_of(b1 * B, B)
            o2 = pl.multiple_of((b1 + 1) * B, B)

            @pl.when(b0 < nb)
            def _():
                gather_wait(o0, 0)

                @pl.when(b1 < nb)
                def _():
                    gather_issue(o1, 1)
                accum(o0, 0)

            @pl.when(b1 < nb)
            def _():
                gather_wait(o1, 1)

                @pl.when(b1 + 1 < nb)
                def _():
                    gather_issue(o2, 0)
                accum(o1, 1)
            return carry
        lax.fori_loop(0, (nb + 1) // 2, bb2, 0)
        left = cnt - nb * B
        base = pl.multiple_of(nb * B, B)
        for t in range(B // 16):
            sl_src = pl.ds(base + t * 16, 16)
            sl_dst = pl.ds(t * 16, 16)
            src_c[sl_dst] = src_c[sl_src]
            loc_c[sl_dst] = loc_c[sl_src]
            w_c[sl_dst] = w_c[sl_src]
        return left

    def pad_tail(cnt):
        for t in range(B // 16):
            sl = pl.ds(cnt + t * 16, 16)
            src_c[sl] = iot + t * 16
            loc_c[sl] = iot + t * 16
            w_c[sl] = z16

    lo0 = tid * RPT
    lo1 = RW + tid * RPT

    def fbody_gen(buf, lo_a, lo_b):
        def fbody(i, carry):
            cnt, cnt2 = carry
            off = pl.multiple_of(i * 16, 16)
            d16 = dst_bufs[buf][pl.ds(off, 16)]
            m0 = (d16 >= lo_a) & (d16 < lo_a + RPT)
            m1 = (d16 >= lo_b) & (d16 < lo_b + RPT)
            s16 = src_bufs[buf][pl.ds(off, 16)] * 2 + c
            wv = w_bufs[buf][pl.ds(off, 16)]
            pos = plsc.cumsum(m0.astype(_i32)) - 1 + cnt
            plsc.store_scatter(src_c, [pos], s16, mask=m0)
            plsc.store_scatter(loc_c, [pos], d16 - lo_a, mask=m0)
            plsc.store_scatter(w_c, [pos], wv, mask=m0)
            pos2 = plsc.cumsum(m1.astype(_i32)) - 1 + cnt2
            plsc.store_scatter(src_c2, [pos2], s16, mask=m1)
            plsc.store_scatter(loc_c2, [pos2], d16 - lo_b, mask=m1)
            plsc.store_scatter(w_c2, [pos2], wv, mask=m1)
            return (cnt + plsc.all_reduce_population_count(m0),
                    cnt2 + plsc.all_reduce_population_count(m1))
        return fbody

    def drain_spill(cnt2, snb):
        nchunks = cnt2 // SPC

        def scw(j, snb_):
            o = pl.multiple_of(j * SPC, 8)
            so = pl.multiple_of(snb_ * SPC, 8)
            pltpu.sync_copy(src_c2.at[pl.ds(o, SPC)],
                            sp_src.at[c, tid, pl.ds(so, SPC)])
            pltpu.sync_copy(loc_c2.at[pl.ds(o, SPC)],
                            sp_loc.at[c, tid, pl.ds(so, SPC)])
            pltpu.sync_copy(w_c2.at[pl.ds(o, SPC)],
                            sp_w.at[c, tid, pl.ds(so, SPC)])
            return snb_ + 1
        snb = lax.fori_loop(0, nchunks, scw, snb)
        base = pl.multiple_of(nchunks * SPC, 8)
        for t in range(SPC // 16):
            sl_src = pl.ds(base + t * 16, 16)
            sl_dst = pl.ds(t * 16, 16)
            src_c2[sl_dst] = src_c2[sl_src]
            loc_c2[sl_dst] = loc_c2[sl_src]
            w_c2[sl_dst] = w_c2[sl_src]
        return cnt2 - nchunks * SPC, snb

    lax.fori_loop(0, RPT, zero_acc, 0)
    fb0 = fbody_gen(0, lo0, lo1)
    fb1 = fbody_gen(1, lo0, lo1)
    zi16 = jnp.zeros((16,), _i32)
    stage_issue(0, 0)

    def gbody2(g2, carry):
        cntv, cnt2v, snb = carry
        g0 = g2 * 2
        g1 = g0 + 1
        stage_wait(g0, 0)
        stage_issue(g1, 1)
        cntv, cnt2v = lax.fori_loop(0, GV, fb0, (cntv, cnt2v))
        cnt = drain_batches(cntv[0])
        cnt2, snb = drain_spill(cnt2v[0], snb)
        cntv = zi16 + cnt
        cnt2v = zi16 + cnt2
        stage_wait(g1, 1)

        @pl.when(g1 + 1 < NGALL)
        def _():
            stage_issue(g1 + 1, 0)
        cntv, cnt2v = lax.fori_loop(0, GV, fb1, (cntv, cnt2v))
        cnt = drain_batches(cntv[0])
        cnt2, snb = drain_spill(cnt2v[0], snb)
        return (zi16 + cnt, zi16 + cnt2, snb)

    cntv, cnt2v, snb = lax.fori_loop(0, NGALL // 2, gbody2, (zi16, zi16, 0))
    cnt = cntv[0]
    cnt2 = cnt2v[0]
    pad_tail(cnt)
    gather_issue(0, 0)
    gather_wait(0, 0)
    accum(0, 0)
    for t in range(SPC // 16):
        sl = pl.ds(cnt2 + t * 16, 16)
        src_c2[sl] = iot + t * 16
        loc_c2[sl] = iot + t * 16
        w_c2[sl] = z16
    so = pl.multiple_of(snb * SPC, 8)
    pltpu.sync_copy(src_c2.at[pl.ds(0, SPC)], sp_src.at[c, tid, pl.ds(so, SPC)])
    pltpu.sync_copy(loc_c2.at[pl.ds(0, SPC)], sp_loc.at[c, tid, pl.ds(so, SPC)])
    pltpu.sync_copy(w_c2.at[pl.ds(0, SPC)], sp_w.at[c, tid, pl.ds(so, SPC)])
    ns = snb + 1
    pltpu.sync_copy(acc, out_hbm.at[c, pl.ds(lo0, RPT)])

    lax.fori_loop(0, RPT, zero_acc, 0)

    def sbody(sidx, carry):
        off = pl.multiple_of(sidx * SPC, 8)
        pltpu.sync_copy(sp_src.at[c, tid, pl.ds(off, SPC)],
                        src_c.at[pl.ds(0, SPC)])
        pltpu.sync_copy(sp_loc.at[c, tid, pl.ds(off, SPC)],
                        loc_c.at[pl.ds(0, SPC)])
        pltpu.sync_copy(sp_w.at[c, tid, pl.ds(off, SPC)],
                        w_c.at[pl.ds(0, SPC)])
        drain_batches(SPC)
        return carry

    lax.fori_loop(0, ns, sbody, 0)
    pltpu.sync_copy(acc, out_hbm.at[c, pl.ds(lo1, RPT)])


def _sc_segment_sum(edge_index, edge_weight, h_flat):
    mesh = plsc.VectorSubcoreMesh(core_axis_name="c", subcore_axis_name="s",
                                  num_cores=NC, num_subcores=NS)
    out = pl.kernel(
        _sc_body,
        out_type=(jax.ShapeDtypeStruct((NC, NR, W), _f32),
                  jax.ShapeDtypeStruct((NC, NS, SPE), _i32),
                  jax.ShapeDtypeStruct((NC, NS, SPE), _i32),
                  jax.ShapeDtypeStruct((NC, NS, SPE), _f32)),
        mesh=mesh,
        compiler_params=pltpu.CompilerParams(needs_layout_passes=False),
        scratch_types=[
            pltpu.VMEM((RPT, W), _f32),
            pltpu.VMEM((G,), _i32),
            pltpu.VMEM((G,), _i32),
            pltpu.VMEM((G,), _i32),
            pltpu.VMEM((G,), _i32),
            pltpu.VMEM((G,), _f32),
            pltpu.VMEM((G,), _f32),
            pltpu.VMEM((CAP,), _i32),
            pltpu.VMEM((CAP,), _i32),
            pltpu.VMEM((CAP,), _f32),
            pltpu.VMEM((CAP2,), _i32),
            pltpu.VMEM((CAP2,), _i32),
            pltpu.VMEM((CAP2,), _f32),
            pltpu.VMEM((B, W), _f32),
            pltpu.VMEM((B, W), _f32),
            pltpu.SemaphoreType.DMA,
            pltpu.SemaphoreType.DMA,
            pltpu.SemaphoreType.DMA,
            pltpu.SemaphoreType.DMA,
        ],
    )(edge_index[0], edge_index[1], edge_weight, h_flat)
    return out[0]


def _comb_body(hf_ref, s_ref, w1_ref, w2_ref, bd_ref, o_ref):
    hb = hf_ref[...].reshape(RB, HP)
    sb = jnp.concatenate([s_ref[0], s_ref[1]], axis=1)
    o_ref[...] = (
        lax.dot_general(sb, w1_ref[...], (((1,), (0,)), ((), ())),
                        preferred_element_type=_f32)
        + lax.dot_general(hb, w2_ref[...], (((1,), (0,)), ((), ())),
                          preferred_element_type=_f32)
        + bd_ref[...])


def _combine(h_flat, s_st, w1, w2, bdec2):
    return pl.pallas_call(
        _comb_body,
        grid=(GRID,),
        in_specs=[
            pl.BlockSpec((2 * RB, W), lambda i: (i, 0)),
            pl.BlockSpec((NC, RB, W), lambda i: (0, i, 0)),
            pl.BlockSpec((HP, D), lambda i: (0, 0)),
            pl.BlockSpec((HP, D), lambda i: (0, 0)),
            pl.BlockSpec((1, D), lambda i: (0, 0)),
        ],
        out_specs=pl.BlockSpec((RB, D), lambda i: (i, 0)),
        out_shape=jax.ShapeDtypeStruct((N, D), _f32),
    )(h_flat, s_st, w1, w2, bdec2)


def kernel(x, edge_index, edge_weight, wenc, benc, w_neigh, w_root, wdec, bdec):
    h_flat = _encoder(x, wenc, benc.reshape(1, H))
    w1, w2 = _fold(w_neigh, w_root, wdec.T)
    s_st = _sc_segment_sum(edge_index, edge_weight, h_flat)
    return _combine(h_flat, s_st, w1, w2, bdec.reshape(1, D))
